# baseline (speedup 1.0000x reference)
import functools

import jax
import jax.numpy as jnp
from jax import lax
from jax.experimental import pallas as pl
from jax.experimental.pallas import tpu as pltpu
from jax.experimental.pallas import tpu_sc as plsc

N = 10000
E = 320000
D_IN = 128
H = 64
C = 40

NC = 2
NS = 16
NW = NC * NS
CHUNK = 128
NBUF = 8
LOOK = 7
NCH = E // CHUNK
BASE = NCH // NW
EXTRA = NCH - BASE * NW
CH = BASE + 2
CH_MAX = 80
NP = 10112
RPS = NP // NS

_mesh = plsc.VectorSubcoreMesh(core_axis_name="c", subcore_axis_name="s")


@functools.partial(
    pl.kernel,
    out_type=jax.ShapeDtypeStruct((NC, NP, 16), jnp.float32),
    mesh=_mesh,
    compiler_params=pltpu.CompilerParams(use_tc_tiling_on_sc=False),
    scratch_types=[
        pltpu.VMEM((CH, CHUNK), jnp.int32),
        pltpu.VMEM((CHUNK, 16), jnp.float32),
        pltpu.VMEM_SHARED((NP, 16), jnp.float32),
        pltpu.SemaphoreType.DMA,
    ],
)
def _deg_sc(edge_hbm, ones_hbm, zeros_hbm, out_hbm, col_v, ones_v, deg_sh, sem):
    c = lax.axis_index("c")
    s = lax.axis_index("s")
    w = c * NS + s
    pltpu.sync_copy(edge_hbm.at[1, pl.ds(w * BASE, BASE)],
                    col_v.at[pl.ds(0, BASE)])

    @pl.when(w < EXTRA)
    def _():
        pltpu.sync_copy(edge_hbm.at[1, pl.ds(NW * BASE + w, 1)],
                        col_v.at[pl.ds(BASE, 1)])

    pltpu.sync_copy(ones_hbm, ones_v)
    pltpu.sync_copy(zeros_hbm.at[pl.ds(s * RPS, RPS)],
                    deg_sh.at[pl.ds(s * RPS, RPS)])
    plsc.subcore_barrier()

    def fire(j, carry):
        pltpu.async_copy(ones_v, deg_sh.at[col_v.at[j]], sem, add=True)
        return carry

    lax.fori_loop(0, BASE, fire, 0)

    @pl.when(w < EXTRA)
    def _():
        pltpu.async_copy(ones_v, deg_sh.at[col_v.at[BASE]], sem, add=True)

    def drain(j, carry):
        pltpu.make_async_copy(ones_v, deg_sh.at[col_v.at[j]], sem).wait()
        return carry

    lax.fori_loop(0, BASE, drain, 0)

    @pl.when(w < EXTRA)
    def _():
        pltpu.make_async_copy(ones_v, deg_sh.at[col_v.at[BASE]], sem).wait()

    plsc.subcore_barrier()
    pltpu.sync_copy(deg_sh.at[pl.ds(s * RPS, RPS)],
                    out_hbm.at[c, pl.ds(s * RPS, RPS)])


@functools.partial(
    pl.kernel,
    out_type=jax.ShapeDtypeStruct((NC, NP, H), jnp.float32),
    mesh=_mesh,
    compiler_params=pltpu.CompilerParams(use_tc_tiling_on_sc=False),
    scratch_types=[
        pltpu.VMEM((CH, CHUNK), jnp.int32),
        pltpu.VMEM((CH, CHUNK), jnp.int32),
        pltpu.VMEM((NBUF, CHUNK, H), jnp.float32),
        pltpu.VMEM_SHARED((NP, H), jnp.float32),
        pltpu.SemaphoreType.DMA((NBUF,)),
        pltpu.SemaphoreType.DMA((NBUF,)),
    ],
)
def _agg_sc(s_hbm, edge_hbm, zeros_hbm, out_hbm,
            row_v, col_v, bufs, agg_sh, gsem, ssem):
    c = lax.axis_index("c")
    s = lax.axis_index("s")
    w = c * NS + s
    nck = BASE + jnp.where(w < EXTRA, 1, 0)
    pltpu.sync_copy(edge_hbm.at[0, pl.ds(w * BASE, BASE)],
                    row_v.at[pl.ds(0, BASE)])
    pltpu.sync_copy(edge_hbm.at[1, pl.ds(w * BASE, BASE)],
                    col_v.at[pl.ds(0, BASE)])

    @pl.when(w < EXTRA)
    def _():
        pltpu.sync_copy(edge_hbm.at[0, pl.ds(NW * BASE + w, 1)],
                        row_v.at[pl.ds(BASE, 1)])
        pltpu.sync_copy(edge_hbm.at[1, pl.ds(NW * BASE + w, 1)],
                        col_v.at[pl.ds(BASE, 1)])

    pltpu.sync_copy(zeros_hbm.at[pl.ds(s * RPS, RPS)],
                    agg_sh.at[pl.ds(s * RPS, RPS)])
    plsc.subcore_barrier()

    def start_gather(j, b):
        pltpu.async_copy(s_hbm.at[row_v.at[j]], bufs.at[b], gsem.at[b])

    def wait_gather(j, b):
        pltpu.make_async_copy(s_hbm.at[row_v.at[j]], bufs.at[b],
                              gsem.at[b]).wait()

    def start_scatter(j, b):
        pltpu.async_copy(bufs.at[b], agg_sh.at[col_v.at[j]], ssem.at[b],
                         add=True)

    def wait_scatter(j, b):
        pltpu.make_async_copy(bufs.at[b], agg_sh.at[col_v.at[j]],
                              ssem.at[b]).wait()

    for j in range(LOOK):
        start_gather(j, j)

    def group(g, carry):
        i0 = g * NBUF
        for b in range(NBUF):
            i = i0 + b
            bn = (b + LOOK) % NBUF
            prev = i + LOOK - NBUF

            @pl.when((prev >= 0) & (prev < nck))
            def _():
                wait_scatter(prev, bn)

            @pl.when(i + LOOK < nck)
            def _():
                start_gather(i + LOOK, bn)

            @pl.when(i < nck)
            def _():
                wait_gather(i, b)
                start_scatter(i, b)
        return carry

    lax.fori_loop(0, CH_MAX // NBUF, group, 0)
    for j in range(CH_MAX + LOOK - NBUF, CH_MAX):

        @pl.when(j < nck)
        def _():
            wait_scatter(j, j % NBUF)

    plsc.subcore_barrier()
    pltpu.sync_copy(agg_sh.at[pl.ds(s * RPS, RPS)],
                    out_hbm.at[c, pl.ds(s * RPS, RPS)])


def _stage1_body(deg_ref, data_ref, w1_ref, s_ref, dinv_ref):
    cnt = deg_ref[0, 0:N, 0:1] + deg_ref[1, 0:N, 0:1]
    dinv = lax.rsqrt(cnt + 1.0)
    h = jnp.dot(data_ref[...], w1_ref[...], preferred_element_type=jnp.float32)
    s_ref[...] = h * dinv
    dinv_ref[...] = dinv


_stage1 = pl.pallas_call(
    _stage1_body,
    out_shape=(jax.ShapeDtypeStruct((N, H), jnp.float32),
               jax.ShapeDtypeStruct((N, 1), jnp.float32)),
)


def _stage2_body(p_ref, s_ref, dinv_ref, b_ref, g_ref, be_ref, w2_ref, out_ref):
    dinv = dinv_ref[...]
    agg = p_ref[0, 0:N, :] + p_ref[1, 0:N, :] + s_ref[...]
    x = jnp.maximum(agg * dinv + b_ref[...], 0.0)
    m = jnp.mean(x, axis=0, keepdims=True)
    v = jnp.mean((x - m) ** 2, axis=0, keepdims=True)
    x = (x - m) * lax.rsqrt(v + 1e-5) * g_ref[...] + be_ref[...]
    x = jnp.maximum(x, 0.0)
    h = jnp.dot(x, w2_ref[...], preferred_element_type=jnp.float32)
    out_ref[...] = h * dinv


_stage2 = pl.pallas_call(
    _stage2_body,
    out_shape=jax.ShapeDtypeStruct((N, H), jnp.float32),
)


def _stage3_body(p_ref, s_ref, dinv_ref, b_ref, g_ref, be_ref, w3_ref, b3_ref,
                 out_ref):
    dinv = dinv_ref[...]
    agg = p_ref[0, 0:N, :] + p_ref[1, 0:N, :] + s_ref[...]
    x = jnp.maximum(agg * dinv + b_ref[...], 0.0)
    m = jnp.mean(x, axis=0, keepdims=True)
    v = jnp.mean((x - m) ** 2, axis=0, keepdims=True)
    x = (x - m) * lax.rsqrt(v + 1e-5) * g_ref[...] + be_ref[...]
    h = jnp.dot(x, w3_ref[...], preferred_element_type=jnp.float32)
    out_ref[...] = jnp.maximum(h + b3_ref[...], 0.0)


_stage3 = pl.pallas_call(
    _stage3_body,
    out_shape=jax.ShapeDtypeStruct((N, C), jnp.float32),
)


def kernel(data, edge_index, W1, b1, g1, be1, W2, b2, g2, be2, W3, b3):
    edge3 = edge_index.reshape(2, NCH, CHUNK)
    ones16 = jnp.zeros((CHUNK, 16), jnp.float32).at[:, 0].set(1.0)
    zeros16 = jnp.zeros((NP, 16), jnp.float32)
    zeros64 = jnp.zeros((NP, H), jnp.float32)

    deg = _deg_sc(edge3, ones16, zeros16)
    s1, dinv = _stage1(deg, data, W1)
    p1 = _agg_sc(s1, edge3, zeros64)
    s2 = _stage2(p1, s1, dinv, b1.reshape(1, H), g1.reshape(1, H),
                 be1.reshape(1, H), W2)
    p2 = _agg_sc(s2, edge3, zeros64)
    out = _stage3(p2, s2, dinv, b2.reshape(1, H), g2.reshape(1, H),
                  be2.reshape(1, H), W3, b3.reshape(1, C))
    return out


# speedup vs baseline: 1.0033x; 1.0033x over previous
import functools

import jax
import jax.numpy as jnp
from jax import lax
from jax.experimental import pallas as pl
from jax.experimental.pallas import tpu as pltpu
from jax.experimental.pallas import tpu_sc as plsc

N = 10000
E = 320000
D_IN = 128
H = 64
C = 40

NC = 2
NS = 16
NW = NC * NS
CHUNK = 128
NBUF = 8
LOOK = 6
NCH = E // CHUNK
BASE = NCH // NW
EXTRA = NCH - BASE * NW
CH = BASE + 2
CH_MAX = 80
NP = 10112
RPS = NP // NS

_mesh = plsc.VectorSubcoreMesh(core_axis_name="c", subcore_axis_name="s")


@functools.partial(
    pl.kernel,
    out_type=jax.ShapeDtypeStruct((NC, NP, 16), jnp.float32),
    mesh=_mesh,
    compiler_params=pltpu.CompilerParams(use_tc_tiling_on_sc=False),
    scratch_types=[
        pltpu.VMEM((CH, CHUNK), jnp.int32),
        pltpu.VMEM((CHUNK, 16), jnp.float32),
        pltpu.VMEM_SHARED((NP, 16), jnp.float32),
        pltpu.SemaphoreType.DMA,
    ],
)
def _deg_sc(edge_hbm, ones_hbm, zeros_hbm, out_hbm, col_v, ones_v, deg_sh, sem):
    c = lax.axis_index("c")
    s = lax.axis_index("s")
    w = c * NS + s
    pltpu.sync_copy(edge_hbm.at[1, pl.ds(w * BASE, BASE)],
                    col_v.at[pl.ds(0, BASE)])

    @pl.when(w < EXTRA)
    def _():
        pltpu.sync_copy(edge_hbm.at[1, pl.ds(NW * BASE + w, 1)],
                        col_v.at[pl.ds(BASE, 1)])

    pltpu.sync_copy(ones_hbm, ones_v)
    pltpu.sync_copy(zeros_hbm.at[pl.ds(s * RPS, RPS)],
                    deg_sh.at[pl.ds(s * RPS, RPS)])
    plsc.subcore_barrier()

    def fire(j, carry):
        pltpu.async_copy(ones_v, deg_sh.at[col_v.at[j]], sem, add=True)
        return carry

    lax.fori_loop(0, BASE, fire, 0)

    @pl.when(w < EXTRA)
    def _():
        pltpu.async_copy(ones_v, deg_sh.at[col_v.at[BASE]], sem, add=True)

    def drain(j, carry):
        pltpu.make_async_copy(ones_v, deg_sh.at[col_v.at[j]], sem).wait()
        return carry

    lax.fori_loop(0, BASE, drain, 0)

    @pl.when(w < EXTRA)
    def _():
        pltpu.make_async_copy(ones_v, deg_sh.at[col_v.at[BASE]], sem).wait()

    plsc.subcore_barrier()
    pltpu.sync_copy(deg_sh.at[pl.ds(s * RPS, RPS)],
                    out_hbm.at[c, pl.ds(s * RPS, RPS)])


@functools.partial(
    pl.kernel,
    out_type=jax.ShapeDtypeStruct((NC, NP, H), jnp.float32),
    mesh=_mesh,
    compiler_params=pltpu.CompilerParams(use_tc_tiling_on_sc=False),
    scratch_types=[
        pltpu.VMEM((CH, CHUNK), jnp.int32),
        pltpu.VMEM((CH, CHUNK), jnp.int32),
        pltpu.VMEM((NBUF, CHUNK, H), jnp.float32),
        pltpu.VMEM_SHARED((NP, H), jnp.float32),
        pltpu.SemaphoreType.DMA((NBUF,)),
        pltpu.SemaphoreType.DMA((NBUF,)),
    ],
)
def _agg_sc(s_hbm, edge_hbm, zeros_hbm, out_hbm,
            row_v, col_v, bufs, agg_sh, gsem, ssem):
    c = lax.axis_index("c")
    s = lax.axis_index("s")
    w = c * NS + s
    nck = BASE + jnp.where(w < EXTRA, 1, 0)
    pltpu.sync_copy(edge_hbm.at[0, pl.ds(w * BASE, BASE)],
                    row_v.at[pl.ds(0, BASE)])
    pltpu.sync_copy(edge_hbm.at[1, pl.ds(w * BASE, BASE)],
                    col_v.at[pl.ds(0, BASE)])

    @pl.when(w < EXTRA)
    def _():
        pltpu.sync_copy(edge_hbm.at[0, pl.ds(NW * BASE + w, 1)],
                        row_v.at[pl.ds(BASE, 1)])
        pltpu.sync_copy(edge_hbm.at[1, pl.ds(NW * BASE + w, 1)],
                        col_v.at[pl.ds(BASE, 1)])

    pltpu.sync_copy(zeros_hbm.at[pl.ds(s * RPS, RPS)],
                    agg_sh.at[pl.ds(s * RPS, RPS)])
    plsc.subcore_barrier()

    def start_gather(j, b):
        pltpu.async_copy(s_hbm.at[row_v.at[j]], bufs.at[b], gsem.at[b])

    def wait_gather(j, b):
        pltpu.make_async_copy(s_hbm.at[row_v.at[j]], bufs.at[b],
                              gsem.at[b]).wait()

    def start_scatter(j, b):
        pltpu.async_copy(bufs.at[b], agg_sh.at[col_v.at[j]], ssem.at[b],
                         add=True)

    def wait_scatter(j, b):
        pltpu.make_async_copy(bufs.at[b], agg_sh.at[col_v.at[j]],
                              ssem.at[b]).wait()

    for j in range(LOOK):
        start_gather(j, j)

    def group(g, carry):
        i0 = g * NBUF
        for b in range(NBUF):
            i = i0 + b
            bn = (b + LOOK) % NBUF
            prev = i + LOOK - NBUF

            @pl.when((prev >= 0) & (prev < nck))
            def _():
                wait_scatter(prev, bn)

            @pl.when(i + LOOK < nck)
            def _():
                start_gather(i + LOOK, bn)

            @pl.when(i < nck)
            def _():
                wait_gather(i, b)
                start_scatter(i, b)
        return carry

    lax.fori_loop(0, CH_MAX // NBUF, group, 0)
    for j in range(CH_MAX + LOOK - NBUF, CH_MAX):

        @pl.when(j < nck)
        def _():
            wait_scatter(j, j % NBUF)

    plsc.subcore_barrier()
    pltpu.sync_copy(agg_sh.at[pl.ds(s * RPS, RPS)],
                    out_hbm.at[c, pl.ds(s * RPS, RPS)])


def _stage1_body(deg_ref, data_ref, w1_ref, s_ref, dinv_ref):
    cnt = deg_ref[0, 0:N, :] + deg_ref[1, 0:N, :]
    dinv = lax.rsqrt(cnt + 1.0)
    h = jnp.dot(data_ref[...], w1_ref[...], preferred_element_type=jnp.float32)
    s_ref[...] = h * dinv
    dinv_ref[...] = dinv


_stage1 = pl.pallas_call(
    _stage1_body,
    out_shape=(jax.ShapeDtypeStruct((N, H), jnp.float32),
               jax.ShapeDtypeStruct((N, 1), jnp.float32)),
)


def _stage2_body(p_ref, s_ref, dinv_ref, b_ref, g_ref, be_ref, w2_ref, out_ref):
    dinv = dinv_ref[...]
    agg = p_ref[0, 0:N, :] + p_ref[1, 0:N, :] + s_ref[...]
    x = jnp.maximum(agg * dinv + b_ref[...], 0.0)
    m = jnp.mean(x, axis=0, keepdims=True)
    v = jnp.mean((x - m) ** 2, axis=0, keepdims=True)
    x = (x - m) * lax.rsqrt(v + 1e-5) * g_ref[...] + be_ref[...]
    x = jnp.maximum(x, 0.0)
    h = jnp.dot(x, w2_ref[...], preferred_element_type=jnp.float32)
    out_ref[...] = h * dinv


_stage2 = pl.pallas_call(
    _stage2_body,
    out_shape=jax.ShapeDtypeStruct((N, H), jnp.float32),
)


def _stage3_body(p_ref, s_ref, dinv_ref, b_ref, g_ref, be_ref, w3_ref, b3_ref,
                 out_ref):
    dinv = dinv_ref[...]
    agg = p_ref[0, 0:N, :] + p_ref[1, 0:N, :] + s_ref[...]
    x = jnp.maximum(agg * dinv + b_ref[...], 0.0)
    m = jnp.mean(x, axis=0, keepdims=True)
    v = jnp.mean((x - m) ** 2, axis=0, keepdims=True)
    x = (x - m) * lax.rsqrt(v + 1e-5) * g_ref[...] + be_ref[...]
    h = jnp.dot(x, w3_ref[...], preferred_element_type=jnp.float32)
    out_ref[...] = jnp.maximum(h + b3_ref[...], 0.0)


_stage3 = pl.pallas_call(
    _stage3_body,
    out_shape=jax.ShapeDtypeStruct((N, C), jnp.float32),
)


def kernel(data, edge_index, W1, b1, g1, be1, W2, b2, g2, be2, W3, b3):
    edge3 = edge_index.reshape(2, NCH, CHUNK)
    ones16 = jnp.zeros((CHUNK, 16), jnp.float32).at[:, 0].set(1.0)
    zeros16 = jnp.zeros((NP, 16), jnp.float32)
    zeros64 = jnp.zeros((NP, H), jnp.float32)

    deg = _deg_sc(edge3, ones16, zeros16)
    s1, dinv = _stage1(deg[:, :, 0:1], data, W1)
    p1 = _agg_sc(s1, edge3, zeros64)
    s2 = _stage2(p1, s1, dinv, b1.reshape(1, H), g1.reshape(1, H),
                 be1.reshape(1, H), W2)
    p2 = _agg_sc(s2, edge3, zeros64)
    out = _stage3(p2, s2, dinv, b2.reshape(1, H), g2.reshape(1, H),
                  be2.reshape(1, H), W3, b3.reshape(1, C))
    return out
